# Initial kernel scaffold; baseline (speedup 1.0000x reference)
import jax
import jax.numpy as jnp
from jax.experimental import pallas as pl


def kernel(node_emb, edge_index, noise, W1, b1, W2, b2, src_idx, dst_idx):
    raise NotImplementedError("write your pallas kernel here")


# speedup vs baseline: 1.7165x; 1.7165x over previous
import functools

import jax
import jax.numpy as jnp
from jax import lax
from jax.experimental import pallas as pl
from jax.experimental.pallas import tpu as pltpu
from jax.experimental.pallas import tpu_sc as plsc

_NC = 2
_NS = 16
_NW = _NC * _NS
_L = 16

_CHUNK = 128


def _mm_body(x_ref, wa_ref, wb_ref, wcd_ref, pa_ref, pb_ref, q_ref):
    x = x_ref[...]
    pa_ref[...] = jnp.dot(x, wa_ref[...], preferred_element_type=jnp.float32)
    pb_ref[...] = jnp.dot(x, wb_ref[...], preferred_element_type=jnp.float32)
    q_ref[...] = jnp.dot(x, wcd_ref[...], preferred_element_type=jnp.float32)


def _tc_tables(node_emb, w1a, w1b, w1cd):
    n, d = node_emb.shape
    h = w1a.shape[1]
    blk = 1000
    grid = n // blk
    return pl.pallas_call(
        _mm_body,
        grid=(grid,),
        in_specs=[
            pl.BlockSpec((blk, d), lambda i: (i, 0)),
            pl.BlockSpec((d, h), lambda i: (0, 0)),
            pl.BlockSpec((d, h), lambda i: (0, 0)),
            pl.BlockSpec((d, 2 * h), lambda i: (0, 0)),
        ],
        out_specs=[
            pl.BlockSpec((blk, h), lambda i: (i, 0)),
            pl.BlockSpec((blk, h), lambda i: (i, 0)),
            pl.BlockSpec((blk, 2 * h), lambda i: (i, 0)),
        ],
        out_shape=[
            jax.ShapeDtypeStruct((n, h), jnp.float32),
            jax.ShapeDtypeStruct((n, h), jnp.float32),
            jax.ShapeDtypeStruct((n, 2 * h), jnp.float32),
        ],
    )(node_emb, w1a, w1b, w1cd)


def _make_sc_stage(n_edges, hidden):
    n_chunks = n_edges // _CHUNK
    n_iters = -(-n_chunks // _NW)
    groups = _CHUNK // _L

    mesh = plsc.VectorSubcoreMesh(
        core_axis_name="c", subcore_axis_name="s",
        num_cores=_NC, num_subcores=_NS,
    )

    @functools.partial(
        pl.kernel,
        out_type=jax.ShapeDtypeStruct((n_edges,), jnp.float32),
        mesh=mesh,
        compiler_params=pltpu.CompilerParams(
            needs_layout_passes=False, use_tc_tiling_on_sc=False),
        scratch_types=[
            pltpu.VMEM((_CHUNK,), jnp.int32),
            pltpu.VMEM((_CHUNK,), jnp.int32),
            pltpu.VMEM((_CHUNK,), jnp.float32),
            pltpu.VMEM((_CHUNK, hidden), jnp.float32),
            pltpu.VMEM((_CHUNK, hidden), jnp.float32),
            pltpu.VMEM((_CHUNK,), jnp.float32),
            pltpu.VMEM((hidden, _L), jnp.float32),
            pltpu.VMEM((hidden, _L), jnp.float32),
            pltpu.VMEM((_L,), jnp.float32),
            pltpu.SemaphoreType.DMA,
            pltpu.SemaphoreType.DMA,
        ],
    )
    def sc_stage(pa_hbm, pb_hbm, col_hbm, row_hbm, noise_hbm, const_hbm,
                 w2_hbm, b2_hbm, out_hbm, colv, rowv, noisev, g1, g2, outv,
                 constv, w2v, b2v, sem1, sem2):
        wid = lax.axis_index("s") * _NC + lax.axis_index("c")
        pltpu.sync_copy(const_hbm, constv)
        pltpu.sync_copy(w2_hbm, w2v)
        pltpu.sync_copy(b2_hbm, b2v)

        def chunk_body(i, carry):
            c = wid + i * _NW

            @pl.when(c < n_chunks)
            def _():
                base = c * _CHUNK
                pltpu.sync_copy(col_hbm.at[pl.ds(base, _CHUNK)], colv)
                pltpu.sync_copy(row_hbm.at[pl.ds(base, _CHUNK)], rowv)
                pltpu.sync_copy(noise_hbm.at[pl.ds(base, _CHUNK)], noisev)
                cp1 = pltpu.async_copy(pa_hbm.at[colv], g1, sem1)
                cp2 = pltpu.async_copy(pb_hbm.at[rowv], g2, sem2)
                cp1.wait()
                cp2.wait()

                def d_body(d, accs):
                    didx = jnp.full((_L,), d, dtype=jnp.int32)
                    cd = constv[d]
                    wd = w2v[d]
                    out = []
                    for g in range(groups):
                        rows = lax.iota(jnp.int32, _L) + (g * _L)
                        v1 = plsc.load_gather(g1, [rows, didx])
                        v2 = plsc.load_gather(g2, [rows, didx])
                        hh = jnp.maximum(v1 + v2 + cd, 0.0)
                        out.append(accs[g] + hh * wd)
                    return tuple(out)

                accs0 = tuple(
                    jnp.zeros((_L,), jnp.float32) for _ in range(groups))
                accs = lax.fori_loop(0, hidden, d_body, accs0)
                b2vec = b2v[...]
                for g in range(groups):
                    nz = noisev[pl.ds(g * _L, _L)]
                    q = (1.0 - nz) / nz
                    w = accs[g] + b2vec
                    outv[pl.ds(g * _L, _L)] = 1.0 / (1.0 + q * jnp.exp(-w))
                pltpu.sync_copy(outv, out_hbm.at[pl.ds(base, _CHUNK)])

            return carry

        lax.fori_loop(0, n_iters, chunk_body, 0)

    return sc_stage


def kernel(node_emb, edge_index, noise, W1, b1, W2, b2, src_idx, dst_idx):
    d = node_emb.shape[1]
    hidden = W2.shape[0]
    n_edges = noise.shape[0]

    w1a = W1[0:d]
    w1b = W1[d:2 * d]
    w1cd = jnp.concatenate([W1[2 * d:3 * d], W1[3 * d:4 * d]], axis=1)

    pa, pb, q = _tc_tables(node_emb, w1a, w1b, w1cd)
    const = q[src_idx, :hidden] + q[dst_idx, hidden:] + b1

    col = edge_index[0]
    row = edge_index[1]
    const_tab = jnp.broadcast_to(const[:, None], (hidden, _L))
    w2_tab = jnp.broadcast_to(W2, (hidden, _L))
    b2v = jnp.broadcast_to(b2, (_L,)).astype(jnp.float32)

    sc_stage = _make_sc_stage(n_edges, hidden)
    return sc_stage(pa, pb, col, row, noise, const_tab, w2_tab, b2v)
